# baseline (speedup 1.0000x reference)
import functools

import jax
import jax.numpy as jnp
from jax import lax
from jax.experimental import pallas as pl
from jax.experimental.pallas import tpu as pltpu
from jax.experimental.pallas import tpu_sc as plsc


def _sc_segment_sum(xP, srcs2, dst2, zA, zC, onesK, *, N, E, H, K):
    info = plsc.get_sparse_core_info()
    NC, NS = info.num_cores, info.num_subcores
    NB = E // K
    bpw = NB // NS
    nx = NB - bpw * NS
    rA = ((N + NS - 1) // NS + 7) // 8 * 8
    rB = N - rA * (NS - 1)
    assert bpw % 2 == 0 and rB > 0 and rB % 8 == 0

    mesh = plsc.VectorSubcoreMesh(core_axis_name="c", subcore_axis_name="s")

    @functools.partial(
        pl.kernel,
        out_type=(jax.ShapeDtypeStruct((NC * N, H), jnp.float32),
                  jax.ShapeDtypeStruct((N, 16), jnp.float32)),
        mesh=mesh,
        scratch_types=[
            pltpu.VMEM((bpw + 1, K), jnp.int32),
            pltpu.VMEM((bpw + 1, K), jnp.int32),
            pltpu.VMEM((K, H), jnp.float32),
            pltpu.VMEM((K, H), jnp.float32),
            pltpu.VMEM((K, 16), jnp.float32),
            pltpu.VMEM_SHARED((N, H), jnp.float32),
            pltpu.VMEM_SHARED((N, 16), jnp.float32),
            pltpu.SemaphoreType.DMA,
            pltpu.SemaphoreType.DMA,
            pltpu.SemaphoreType.DMA,
            pltpu.SemaphoreType.DMA,
            pltpu.SemaphoreType.DMA,
        ],
        compiler_params=pltpu.CompilerParams(use_tc_tiling_on_sc=False),
    )
    def sc_agg(xp_hbm, src_hbm, dst_hbm, za_hbm, zc_hbm, ones_hbm,
               out_hbm, cnt_hbm,
               si, di, rows0, rows1, onesv, acc, cnt,
               gsem0, gsem1, ssem0, ssem1, csem):
        c = lax.axis_index("c")
        s = lax.axis_index("s")

        @pl.when(s < NS - 1)
        def _():
            pltpu.sync_copy(za_hbm.at[pl.ds(0, rA)],
                            acc.at[pl.ds(s * rA, rA)])

        @pl.when(s == NS - 1)
        def _():
            pltpu.sync_copy(za_hbm.at[pl.ds(0, rB)],
                            acc.at[pl.ds((NS - 1) * rA, rB)])

        @pl.when((c == 0) & (s < NS - 1))
        def _():
            pltpu.sync_copy(zc_hbm.at[pl.ds(0, rA)],
                            cnt.at[pl.ds(s * rA, rA)])

        @pl.when((c == 0) & (s == NS - 1))
        def _():
            pltpu.sync_copy(zc_hbm.at[pl.ds(0, rB)],
                            cnt.at[pl.ds((NS - 1) * rA, rB)])

        @pl.when(c == 0)
        def _():
            pltpu.sync_copy(ones_hbm, onesv)

        b0 = s * bpw
        pltpu.sync_copy(src_hbm.at[pl.ds(b0, bpw)], si.at[pl.ds(0, bpw)])
        pltpu.sync_copy(dst_hbm.at[pl.ds(b0, bpw)], di.at[pl.ds(0, bpw)])

        @pl.when(s < nx)
        def _():
            xb = NB - nx + s
            pltpu.sync_copy(src_hbm.at[pl.ds(xb, 1)], si.at[pl.ds(bpw, 1)])
            pltpu.sync_copy(dst_hbm.at[pl.ds(xb, 1)], di.at[pl.ds(bpw, 1)])

        def remap(jb, carry):
            for t in range(K // 16):
                v = si[jb, pl.ds(t * 16, 16)]
                si[jb, pl.ds(t * 16, 16)] = v * 2 + c
            return carry

        lax.fori_loop(0, bpw + 1, remap, 0, unroll=False)

        plsc.subcore_barrier()

        rows = (rows0, rows1)
        gsem = (gsem0, gsem1)
        ssem = (ssem0, ssem1)

        pltpu.async_copy(xp_hbm.at[si.at[0]], rows0, gsem0)

        def step(j, k):
            @pl.when(j >= 1)
            def _():
                pltpu.make_async_copy(
                    rows[1 - k], acc.at[di.at[0]], ssem[1 - k]).wait()

            @pl.when(j + 1 < bpw)
            def _():
                pltpu.async_copy(xp_hbm.at[si.at[j + 1]], rows[1 - k],
                                 gsem[1 - k])

            pltpu.make_async_copy(xp_hbm.at[si.at[0]], rows[k],
                                  gsem[k]).wait()
            pltpu.async_copy(rows[k], acc.at[di.at[j]], ssem[k], add=True)

            @pl.when((c == 0) & (j >= 1))
            def _():
                pltpu.make_async_copy(onesv, cnt.at[di.at[0]], csem).wait()

            @pl.when(c == 0)
            def _():
                pltpu.async_copy(onesv, cnt.at[di.at[j]], csem, add=True)

        def body(g, carry):
            step(2 * g, 0)
            step(2 * g + 1, 1)
            return carry

        lax.fori_loop(0, bpw // 2, body, 0, unroll=False)

        pltpu.make_async_copy(rows1, acc.at[di.at[0]], ssem1).wait()

        @pl.when(c == 0)
        def _():
            pltpu.make_async_copy(onesv, cnt.at[di.at[0]], csem).wait()

        @pl.when(s < nx)
        def _():
            pltpu.async_copy(xp_hbm.at[si.at[bpw]], rows0, gsem0).wait()
            pltpu.async_copy(rows0, acc.at[di.at[bpw]], ssem0,
                             add=True).wait()

            @pl.when(c == 0)
            def _():
                pltpu.async_copy(onesv, cnt.at[di.at[bpw]], csem,
                                 add=True).wait()

        plsc.subcore_barrier()

        @pl.when(s < NS - 1)
        def _():
            pltpu.sync_copy(acc.at[pl.ds(s * rA, rA)],
                            out_hbm.at[pl.ds(c * N + s * rA, rA)])

        @pl.when(s == NS - 1)
        def _():
            pltpu.sync_copy(acc.at[pl.ds((NS - 1) * rA, rB)],
                            out_hbm.at[pl.ds(c * N + (NS - 1) * rA, rB)])

        @pl.when((c == 0) & (s < NS - 1))
        def _():
            pltpu.sync_copy(cnt.at[pl.ds(s * rA, rA)],
                            cnt_hbm.at[pl.ds(s * rA, rA)])

        @pl.when((c == 0) & (s == NS - 1))
        def _():
            pltpu.sync_copy(cnt.at[pl.ds((NS - 1) * rA, rB)],
                            cnt_hbm.at[pl.ds((NS - 1) * rA, rB)])

    return sc_agg(xP, srcs2, dst2, zA, zC, onesK)


def _tc_dense(x, agg2, cnt, W_self, WnT, WnB, b2, *, N, D, B):
    H = D // 2

    def body(x_ref, al_ref, ar_ref, cnt_ref, ws_ref, wt_ref, wb_ref,
             b_ref, out_ref):
        r = 1.0 / jnp.maximum(cnt_ref[:, :1], 1.0)
        dn = (((1,), (0,)), ((), ()))
        acc = lax.dot_general(x_ref[...], ws_ref[...], dn,
                              precision=lax.Precision.HIGHEST,
                              preferred_element_type=jnp.float32)
        acc += lax.dot_general(al_ref[...] * r, wt_ref[...], dn,
                               precision=lax.Precision.HIGHEST,
                               preferred_element_type=jnp.float32)
        acc += lax.dot_general(ar_ref[...] * r, wb_ref[...], dn,
                               precision=lax.Precision.HIGHEST,
                               preferred_element_type=jnp.float32)
        out_ref[...] = acc + b_ref[...]

    nb = N // B
    return pl.pallas_call(
        body,
        grid=(nb,),
        in_specs=[
            pl.BlockSpec((B, D), lambda i: (i, 0)),
            pl.BlockSpec((B, H), lambda i: (i, 0)),
            pl.BlockSpec((B, H), lambda i: (nb + i, 0)),
            pl.BlockSpec((B, 16), lambda i: (i, 0)),
            pl.BlockSpec((D, D), lambda i: (0, 0)),
            pl.BlockSpec((H, D), lambda i: (0, 0)),
            pl.BlockSpec((H, D), lambda i: (0, 0)),
            pl.BlockSpec((1, D), lambda i: (0, 0)),
        ],
        out_specs=pl.BlockSpec((B, D), lambda i: (i, 0)),
        out_shape=jax.ShapeDtypeStruct((N, D), jnp.float32),
    )(x, agg2, agg2, cnt, W_self, WnT, WnB, b2)


def kernel(x, edge_index, W_self, W_neigh, b):
    N, D = x.shape
    E = edge_index.shape[1]
    H = D // 2
    K = 64
    NS = 16
    rA = ((N + NS - 1) // NS + 7) // 8 * 8

    xP = x.reshape(2 * N, H)
    srcs2 = edge_index[0].reshape(E // K, K)
    dst2 = edge_index[1].reshape(E // K, K)
    zA = jnp.zeros((rA, H), jnp.float32)
    zC = jnp.zeros((rA, 16), jnp.float32)
    onesK = jnp.ones((K, 16), jnp.float32)

    agg2, cnt = _sc_segment_sum(xP, srcs2, dst2, zA, zC, onesK,
                                N=N, E=E, H=H, K=K)

    WnT = W_neigh[:H, :]
    WnB = W_neigh[H:, :]
    b2 = b.reshape(1, D)

    return _tc_dense(x, agg2, cnt, W_self, WnT, WnB, b2, N=N, D=D, B=2000)


# speedup vs baseline: 1.0649x; 1.0649x over previous
import functools

import jax
import jax.numpy as jnp
from jax import lax
from jax.experimental import pallas as pl
from jax.experimental.pallas import tpu as pltpu
from jax.experimental.pallas import tpu_sc as plsc


def _sc_segment_sum(xP, srcs2, dst2, zA, zC, onesK, *, N, E, H, K):
    info = plsc.get_sparse_core_info()
    NC, NS = info.num_cores, info.num_subcores
    NB = E // K
    bpw = NB // NS
    nx = NB - bpw * NS
    rA = ((N + NS - 1) // NS + 7) // 8 * 8
    rB = N - rA * (NS - 1)
    assert bpw % 2 == 0 and rB > 0 and rB % 8 == 0

    mesh = plsc.VectorSubcoreMesh(core_axis_name="c", subcore_axis_name="s")

    @functools.partial(
        pl.kernel,
        out_type=(jax.ShapeDtypeStruct((NC * N, H), jnp.float32),
                  jax.ShapeDtypeStruct((N, 16), jnp.float32)),
        mesh=mesh,
        scratch_types=[
            pltpu.VMEM((bpw + 1, K), jnp.int32),
            pltpu.VMEM((bpw + 1, K), jnp.int32),
            pltpu.VMEM((K, H), jnp.float32),
            pltpu.VMEM((K, H), jnp.float32),
            pltpu.VMEM((K, 16), jnp.float32),
            pltpu.VMEM_SHARED((N, H), jnp.float32),
            pltpu.VMEM_SHARED((N, 16), jnp.float32),
            pltpu.SemaphoreType.DMA,
            pltpu.SemaphoreType.DMA,
            pltpu.SemaphoreType.DMA,
            pltpu.SemaphoreType.DMA,
            pltpu.SemaphoreType.DMA,
        ],
        compiler_params=pltpu.CompilerParams(use_tc_tiling_on_sc=False),
    )
    def sc_agg(xp_hbm, src_hbm, dst_hbm, za_hbm, zc_hbm, ones_hbm,
               out_hbm, cnt_hbm,
               si, di, rows0, rows1, onesv, acc, cnt,
               gsem0, gsem1, ssem0, ssem1, csem):
        c = lax.axis_index("c")
        s = lax.axis_index("s")

        @pl.when(s < NS - 1)
        def _():
            pltpu.sync_copy(za_hbm.at[pl.ds(0, rA)],
                            acc.at[pl.ds(s * rA, rA)])

        @pl.when(s == NS - 1)
        def _():
            pltpu.sync_copy(za_hbm.at[pl.ds(0, rB)],
                            acc.at[pl.ds((NS - 1) * rA, rB)])

        @pl.when((c == 0) & (s < NS - 1))
        def _():
            pltpu.sync_copy(zc_hbm.at[pl.ds(0, rA)],
                            cnt.at[pl.ds(s * rA, rA)])

        @pl.when((c == 0) & (s == NS - 1))
        def _():
            pltpu.sync_copy(zc_hbm.at[pl.ds(0, rB)],
                            cnt.at[pl.ds((NS - 1) * rA, rB)])

        @pl.when(c == 0)
        def _():
            pltpu.sync_copy(ones_hbm, onesv)

        b0 = s * bpw
        pltpu.sync_copy(src_hbm.at[pl.ds(b0, bpw)], si.at[pl.ds(0, bpw)])
        pltpu.sync_copy(dst_hbm.at[pl.ds(b0, bpw)], di.at[pl.ds(0, bpw)])

        @pl.when(s < nx)
        def _():
            xb = NB - nx + s
            pltpu.sync_copy(src_hbm.at[pl.ds(xb, 1)], si.at[pl.ds(bpw, 1)])
            pltpu.sync_copy(dst_hbm.at[pl.ds(xb, 1)], di.at[pl.ds(bpw, 1)])

        def remap(jb, carry):
            for t in range(K // 16):
                v = si[jb, pl.ds(t * 16, 16)]
                si[jb, pl.ds(t * 16, 16)] = v * 2 + c
            return carry

        lax.fori_loop(0, bpw + 1, remap, 0, unroll=False)

        plsc.subcore_barrier()

        rows = (rows0, rows1)
        gsem = (gsem0, gsem1)
        ssem = (ssem0, ssem1)

        pltpu.async_copy(xp_hbm.at[si.at[0]], rows0, gsem0)

        def step(j, k):
            @pl.when(j >= 1)
            def _():
                pltpu.make_async_copy(
                    rows[1 - k], acc.at[di.at[0]], ssem[1 - k]).wait()

            @pl.when(j + 1 < bpw)
            def _():
                pltpu.async_copy(xp_hbm.at[si.at[j + 1]], rows[1 - k],
                                 gsem[1 - k])

            pltpu.make_async_copy(xp_hbm.at[si.at[0]], rows[k],
                                  gsem[k]).wait()
            pltpu.async_copy(rows[k], acc.at[di.at[j]], ssem[k], add=True)

            @pl.when((c == 0) & (j >= 1))
            def _():
                pltpu.make_async_copy(onesv, cnt.at[di.at[0]], csem).wait()

            @pl.when(c == 0)
            def _():
                pltpu.async_copy(onesv, cnt.at[di.at[j]], csem, add=True)

        def body(g, carry):
            step(2 * g, 0)
            step(2 * g + 1, 1)
            return carry

        lax.fori_loop(0, bpw // 2, body, 0, unroll=False)

        pltpu.make_async_copy(rows1, acc.at[di.at[0]], ssem1).wait()

        @pl.when(c == 0)
        def _():
            pltpu.make_async_copy(onesv, cnt.at[di.at[0]], csem).wait()

        @pl.when(s < nx)
        def _():
            pltpu.async_copy(xp_hbm.at[si.at[bpw]], rows0, gsem0).wait()
            pltpu.async_copy(rows0, acc.at[di.at[bpw]], ssem0,
                             add=True).wait()

            @pl.when(c == 0)
            def _():
                pltpu.async_copy(onesv, cnt.at[di.at[bpw]], csem,
                                 add=True).wait()

        plsc.subcore_barrier()

        @pl.when(s < NS - 1)
        def _():
            pltpu.sync_copy(acc.at[pl.ds(s * rA, rA)],
                            out_hbm.at[pl.ds(c * N + s * rA, rA)])

        @pl.when(s == NS - 1)
        def _():
            pltpu.sync_copy(acc.at[pl.ds((NS - 1) * rA, rB)],
                            out_hbm.at[pl.ds(c * N + (NS - 1) * rA, rB)])

        @pl.when((c == 0) & (s < NS - 1))
        def _():
            pltpu.sync_copy(cnt.at[pl.ds(s * rA, rA)],
                            cnt_hbm.at[pl.ds(s * rA, rA)])

        @pl.when((c == 0) & (s == NS - 1))
        def _():
            pltpu.sync_copy(cnt.at[pl.ds((NS - 1) * rA, rB)],
                            cnt_hbm.at[pl.ds((NS - 1) * rA, rB)])

    return sc_agg(xP, srcs2, dst2, zA, zC, onesK)


def _tc_dense(x, agg2, cnt, W_self, WnT, WnB, b2, *, N, D, B):
    H = D // 2

    def dot3(a, w):
        ah = a.astype(jnp.bfloat16)
        al = (a - ah.astype(jnp.float32)).astype(jnp.bfloat16)
        wh = w.astype(jnp.bfloat16)
        wl = (w - wh.astype(jnp.float32)).astype(jnp.bfloat16)
        dn = (((1,), (0,)), ((), ()))
        f = functools.partial(lax.dot_general, dimension_numbers=dn,
                              preferred_element_type=jnp.float32)
        return f(ah, wh) + f(al, wh) + f(ah, wl)

    def body(x_ref, al_ref, ar_ref, cnt_ref, ws_ref, wt_ref, wb_ref,
             b_ref, out_ref):
        r = 1.0 / jnp.maximum(cnt_ref[:, :1], 1.0)
        acc = dot3(x_ref[...], ws_ref[...])
        acc += dot3(al_ref[...] * r, wt_ref[...])
        acc += dot3(ar_ref[...] * r, wb_ref[...])
        out_ref[...] = acc + b_ref[...]

    nb = N // B
    return pl.pallas_call(
        body,
        grid=(nb,),
        in_specs=[
            pl.BlockSpec((B, D), lambda i: (i, 0)),
            pl.BlockSpec((B, H), lambda i: (i, 0)),
            pl.BlockSpec((B, H), lambda i: (nb + i, 0)),
            pl.BlockSpec((B, 16), lambda i: (i, 0)),
            pl.BlockSpec((D, D), lambda i: (0, 0)),
            pl.BlockSpec((H, D), lambda i: (0, 0)),
            pl.BlockSpec((H, D), lambda i: (0, 0)),
            pl.BlockSpec((1, D), lambda i: (0, 0)),
        ],
        out_specs=pl.BlockSpec((B, D), lambda i: (i, 0)),
        out_shape=jax.ShapeDtypeStruct((N, D), jnp.float32),
    )(x, agg2, agg2, cnt, W_self, WnT, WnB, b2)


def kernel(x, edge_index, W_self, W_neigh, b):
    N, D = x.shape
    E = edge_index.shape[1]
    H = D // 2
    K = 64
    NS = 16
    rA = ((N + NS - 1) // NS + 7) // 8 * 8

    xP = x.reshape(2 * N, H)
    srcs2 = edge_index[0].reshape(E // K, K)
    dst2 = edge_index[1].reshape(E // K, K)
    zA = jnp.zeros((rA, H), jnp.float32)
    zC = jnp.zeros((rA, 16), jnp.float32)
    onesK = jnp.ones((K, 16), jnp.float32)

    agg2, cnt = _sc_segment_sum(xP, srcs2, dst2, zA, zC, onesK,
                                N=N, E=E, H=H, K=K)

    WnT = W_neigh[:H, :]
    WnB = W_neigh[H:, :]
    b2 = b.reshape(1, D)

    return _tc_dense(x, agg2, cnt, W_self, WnT, WnB, b2, N=N, D=D, B=2000)


# speedup vs baseline: 1.0888x; 1.0225x over previous
import functools

import jax
import jax.numpy as jnp
from jax import lax
from jax.experimental import pallas as pl
from jax.experimental.pallas import tpu as pltpu
from jax.experimental.pallas import tpu_sc as plsc


def _sc_segment_sum(xP, srcs2, dst2, zA, zC, onesK, *, N, E, H, K):
    info = plsc.get_sparse_core_info()
    NC, NS = info.num_cores, info.num_subcores
    NB = E // K
    bpw = NB // NS
    nx = NB - bpw * NS
    rA = ((N + NS - 1) // NS + 7) // 8 * 8
    rB = N - rA * (NS - 1)
    assert bpw % 2 == 0 and rB > 0 and rB % 8 == 0

    mesh = plsc.VectorSubcoreMesh(core_axis_name="c", subcore_axis_name="s")

    @functools.partial(
        pl.kernel,
        out_type=(jax.ShapeDtypeStruct((NC * N, H), jnp.float32),
                  jax.ShapeDtypeStruct((N, 16), jnp.float32)),
        mesh=mesh,
        scratch_types=[
            pltpu.VMEM((bpw + 1, K), jnp.int32),
            pltpu.VMEM((bpw + 1, K), jnp.int32),
            pltpu.VMEM((K, H), jnp.float32),
            pltpu.VMEM((K, H), jnp.float32),
            pltpu.VMEM((K, 16), jnp.float32),
            pltpu.VMEM_SHARED((N, H), jnp.float32),
            pltpu.VMEM_SHARED((N, 16), jnp.float32),
            pltpu.SemaphoreType.DMA,
            pltpu.SemaphoreType.DMA,
            pltpu.SemaphoreType.DMA,
            pltpu.SemaphoreType.DMA,
            pltpu.SemaphoreType.DMA,
        ],
        compiler_params=pltpu.CompilerParams(use_tc_tiling_on_sc=False),
    )
    def sc_agg(xp_hbm, src_hbm, dst_hbm, za_hbm, zc_hbm, ones_hbm,
               out_hbm, cnt_hbm,
               si, di, rows0, rows1, onesv, acc, cnt,
               gsem0, gsem1, ssem0, ssem1, csem):
        c = lax.axis_index("c")
        s = lax.axis_index("s")

        r0 = s * rA
        azeroA = pltpu.make_async_copy(za_hbm.at[pl.ds(0, rA)],
                                       acc.at[pl.ds(r0, rA)], ssem0)
        azeroB = pltpu.make_async_copy(za_hbm.at[pl.ds(0, rB)],
                                       acc.at[pl.ds((NS - 1) * rA, rB)],
                                       ssem0)

        @pl.when(s < NS - 1)
        def _():
            azeroA.start()

        @pl.when(s == NS - 1)
        def _():
            azeroB.start()

        b0 = s * bpw
        pltpu.async_copy(src_hbm.at[pl.ds(b0, bpw)], si.at[pl.ds(0, bpw)],
                         gsem0)
        pltpu.async_copy(dst_hbm.at[pl.ds(b0, bpw)], di.at[pl.ds(0, bpw)],
                         gsem1)

        czeroA = pltpu.make_async_copy(zc_hbm.at[pl.ds(0, rA)],
                                       cnt.at[pl.ds(r0, rA)], csem)
        czeroB = pltpu.make_async_copy(zc_hbm.at[pl.ds(0, rB)],
                                       cnt.at[pl.ds((NS - 1) * rA, rB)],
                                       csem)

        @pl.when((c == 0) & (s < NS - 1))
        def _():
            czeroA.start()

        @pl.when((c == 0) & (s == NS - 1))
        def _():
            czeroB.start()

        @pl.when(c == 0)
        def _():
            pltpu.sync_copy(ones_hbm, onesv)

        pltpu.make_async_copy(src_hbm.at[pl.ds(b0, bpw)],
                              si.at[pl.ds(0, bpw)], gsem0).wait()
        pltpu.make_async_copy(dst_hbm.at[pl.ds(b0, bpw)],
                              di.at[pl.ds(0, bpw)], gsem1).wait()

        @pl.when(s < nx)
        def _():
            xb = NB - nx + s
            pltpu.sync_copy(src_hbm.at[pl.ds(xb, 1)], si.at[pl.ds(bpw, 1)])
            pltpu.sync_copy(dst_hbm.at[pl.ds(xb, 1)], di.at[pl.ds(bpw, 1)])

        def remap(jb, carry):
            for t in range(K // 16):
                v = si[jb, pl.ds(t * 16, 16)]
                si[jb, pl.ds(t * 16, 16)] = v * 2 + c
            return carry

        lax.fori_loop(0, bpw + 1, remap, 0, unroll=False)

        @pl.when(s < NS - 1)
        def _():
            azeroA.wait()

        @pl.when(s == NS - 1)
        def _():
            azeroB.wait()

        @pl.when((c == 0) & (s < NS - 1))
        def _():
            czeroA.wait()

        @pl.when((c == 0) & (s == NS - 1))
        def _():
            czeroB.wait()

        plsc.subcore_barrier()

        rows = (rows0, rows1)
        gsem = (gsem0, gsem1)
        ssem = (ssem0, ssem1)

        pltpu.async_copy(xp_hbm.at[si.at[0]], rows0, gsem0)

        def step(j, k):
            @pl.when(j >= 1)
            def _():
                pltpu.make_async_copy(
                    rows[1 - k], acc.at[di.at[0]], ssem[1 - k]).wait()

            @pl.when(j + 1 < bpw)
            def _():
                pltpu.async_copy(xp_hbm.at[si.at[j + 1]], rows[1 - k],
                                 gsem[1 - k])

            pltpu.make_async_copy(xp_hbm.at[si.at[0]], rows[k],
                                  gsem[k]).wait()
            pltpu.async_copy(rows[k], acc.at[di.at[j]], ssem[k], add=True)

            @pl.when((c == 0) & (j >= 1))
            def _():
                pltpu.make_async_copy(onesv, cnt.at[di.at[0]], csem).wait()

            @pl.when(c == 0)
            def _():
                pltpu.async_copy(onesv, cnt.at[di.at[j]], csem, add=True)

        def body(g, carry):
            step(2 * g, 0)
            step(2 * g + 1, 1)
            return carry

        lax.fori_loop(0, bpw // 2, body, 0, unroll=False)

        pltpu.make_async_copy(rows1, acc.at[di.at[0]], ssem1).wait()

        @pl.when(c == 0)
        def _():
            pltpu.make_async_copy(onesv, cnt.at[di.at[0]], csem).wait()

        @pl.when(s < nx)
        def _():
            pltpu.async_copy(xp_hbm.at[si.at[bpw]], rows0, gsem0).wait()
            pltpu.async_copy(rows0, acc.at[di.at[bpw]], ssem0,
                             add=True).wait()

            @pl.when(c == 0)
            def _():
                pltpu.async_copy(onesv, cnt.at[di.at[bpw]], csem,
                                 add=True).wait()

        plsc.subcore_barrier()

        @pl.when(s < NS - 1)
        def _():
            pltpu.sync_copy(acc.at[pl.ds(s * rA, rA)],
                            out_hbm.at[pl.ds(c * N + s * rA, rA)])

        @pl.when(s == NS - 1)
        def _():
            pltpu.sync_copy(acc.at[pl.ds((NS - 1) * rA, rB)],
                            out_hbm.at[pl.ds(c * N + (NS - 1) * rA, rB)])

        @pl.when((c == 0) & (s < NS - 1))
        def _():
            pltpu.sync_copy(cnt.at[pl.ds(s * rA, rA)],
                            cnt_hbm.at[pl.ds(s * rA, rA)])

        @pl.when((c == 0) & (s == NS - 1))
        def _():
            pltpu.sync_copy(cnt.at[pl.ds((NS - 1) * rA, rB)],
                            cnt_hbm.at[pl.ds((NS - 1) * rA, rB)])

    return sc_agg(xP, srcs2, dst2, zA, zC, onesK)


def _tc_dense(x, agg2, cnt, W_self, WnT, WnB, b2, *, N, D, B):
    H = D // 2

    def dot3(a, w):
        ah = a.astype(jnp.bfloat16)
        al = (a - ah.astype(jnp.float32)).astype(jnp.bfloat16)
        wh = w.astype(jnp.bfloat16)
        wl = (w - wh.astype(jnp.float32)).astype(jnp.bfloat16)
        dn = (((1,), (0,)), ((), ()))
        f = functools.partial(lax.dot_general, dimension_numbers=dn,
                              preferred_element_type=jnp.float32)
        return f(ah, wh) + f(al, wh) + f(ah, wl)

    def body(x_ref, al_ref, ar_ref, cnt_ref, ws_ref, wt_ref, wb_ref,
             b_ref, out_ref):
        r = 1.0 / jnp.maximum(cnt_ref[:, :1], 1.0)
        acc = dot3(x_ref[...], ws_ref[...])
        acc += dot3(al_ref[...] * r, wt_ref[...])
        acc += dot3(ar_ref[...] * r, wb_ref[...])
        out_ref[...] = acc + b_ref[...]

    nb = N // B
    return pl.pallas_call(
        body,
        grid=(nb,),
        in_specs=[
            pl.BlockSpec((B, D), lambda i: (i, 0)),
            pl.BlockSpec((B, H), lambda i: (i, 0)),
            pl.BlockSpec((B, H), lambda i: (nb + i, 0)),
            pl.BlockSpec((B, 16), lambda i: (i, 0)),
            pl.BlockSpec((D, D), lambda i: (0, 0)),
            pl.BlockSpec((H, D), lambda i: (0, 0)),
            pl.BlockSpec((H, D), lambda i: (0, 0)),
            pl.BlockSpec((1, D), lambda i: (0, 0)),
        ],
        out_specs=pl.BlockSpec((B, D), lambda i: (i, 0)),
        out_shape=jax.ShapeDtypeStruct((N, D), jnp.float32),
    )(x, agg2, agg2, cnt, W_self, WnT, WnB, b2)


def kernel(x, edge_index, W_self, W_neigh, b):
    N, D = x.shape
    E = edge_index.shape[1]
    H = D // 2
    K = 64
    NS = 16
    rA = ((N + NS - 1) // NS + 7) // 8 * 8

    xP = x.reshape(2 * N, H)
    srcs2 = edge_index[0].reshape(E // K, K)
    dst2 = edge_index[1].reshape(E // K, K)
    zA = jnp.zeros((rA, H), jnp.float32)
    zC = jnp.zeros((rA, 16), jnp.float32)
    onesK = jnp.ones((K, 16), jnp.float32)

    agg2, cnt = _sc_segment_sum(xP, srcs2, dst2, zA, zC, onesK,
                                N=N, E=E, H=H, K=K)

    WnT = W_neigh[:H, :]
    WnB = W_neigh[H:, :]
    b2 = b.reshape(1, D)

    return _tc_dense(x, agg2, cnt, W_self, WnT, WnB, b2, N=N, D=D, B=2000)
